# baseline (speedup 1.0000x reference)
import functools

import jax
import jax.numpy as jnp
from jax import lax
from jax.experimental import pallas as pl
from jax.experimental.pallas import tpu as pltpu
from jax.experimental.pallas import tpu_sc as plsc

_B, _L, _D = 64, 576, 768
_NC, _NS = 2, 16
_G, _J = 4, 8
_BPG = _B // _G
_RPW = _L // _J
_CHR = 24
_NCH = _RPW // _CHR
_NVC = _D // 16


def _sc_body(in_hbm, pos_hbm, cls_hbm, out_hbm, pos_v, buf_v, cls_v, tmp_v):
    wid = lax.axis_index("s") * _NC + lax.axis_index("c")
    g = wid // _J
    j = wid % _J
    b0 = g * _BPG

    def chunk_body(ch, carry):
        r0 = j * _RPW + ch * _CHR
        pltpu.sync_copy(pos_hbm.at[pl.ds(r0, _CHR)], pos_v)

        def batch_body(b, carry2):
            bb = b0 + b
            pltpu.sync_copy(in_hbm.at[bb, pl.ds(r0, _CHR)], buf_v)

            def row_body(r, carry3):
                def col_body(c, carry4):
                    off = c * 16
                    buf_v[r, pl.ds(off, 16)] = (
                        buf_v[r, pl.ds(off, 16)] + pos_v[r, pl.ds(off, 16)]
                    )
                    return carry4

                return lax.fori_loop(0, _NVC, col_body, carry3)

            lax.fori_loop(0, _CHR, row_body, carry2)
            pltpu.sync_copy(buf_v, out_hbm.at[bb, pl.ds(r0, _CHR)])
            return carry2

        lax.fori_loop(0, _BPG, batch_body, carry)
        return carry

    lax.fori_loop(0, _NCH, chunk_body, 0)

    @pl.when(j == 0)
    def _():
        pltpu.sync_copy(cls_hbm.at[0], cls_v)
        pltpu.sync_copy(pos_hbm.at[pl.ds(_L, 1)], tmp_v)

        def cls_col(c, carry):
            off = c * 16
            cls_v[0, pl.ds(off, 16)] = (
                cls_v[0, pl.ds(off, 16)] + tmp_v[0, pl.ds(off, 16)]
            )
            return carry

        lax.fori_loop(0, _NVC, cls_col, 0)

        def cls_batch(b, carry):
            pltpu.sync_copy(cls_v, out_hbm.at[b0 + b, pl.ds(_L, 1)])
            return carry

        lax.fori_loop(0, _BPG, cls_batch, 0)


@functools.partial(
    pl.kernel,
    mesh=plsc.VectorSubcoreMesh(core_axis_name="c", subcore_axis_name="s"),
    out_type=jax.ShapeDtypeStruct((_B, _L + 1, _D), jnp.float32),
    scratch_types=[
        pltpu.VMEM((_CHR, _D), jnp.float32),
        pltpu.VMEM((_CHR, _D), jnp.float32),
        pltpu.VMEM((1, _D), jnp.float32),
        pltpu.VMEM((1, _D), jnp.float32),
    ],
)
def _sc_kernel(in_hbm, pos_hbm, cls_hbm, out_hbm, pos_v, buf_v, cls_v, tmp_v):
    _sc_body(in_hbm, pos_hbm, cls_hbm, out_hbm, pos_v, buf_v, cls_v, tmp_v)


def kernel(inputs, pos_table, class_token):
    return _sc_kernel(inputs, pos_table, class_token)


# speedup vs baseline: 1.6904x; 1.6904x over previous
import functools

import jax
import jax.numpy as jnp
from jax import lax
from jax.experimental import pallas as pl
from jax.experimental.pallas import tpu as pltpu
from jax.experimental.pallas import tpu_sc as plsc

_B, _L, _D = 64, 576, 768
_NC, _NS = 2, 16
_G, _J = 4, 8
_BPG = _B // _G
_RPW = _L // _J
_CHR = 24
_NCH = _RPW // _CHR
_NVC = _D // 16


def _sc_body(in_hbm, pos_hbm, cls_hbm, out_hbm, pos_v, buf_v, cls_v, tmp_v):
    wid = lax.axis_index("s") * _NC + lax.axis_index("c")
    g = wid // _J
    j = wid % _J
    b0 = g * _BPG

    def chunk_body(ch, carry):
        r0 = j * _RPW + ch * _CHR
        pltpu.sync_copy(pos_hbm.at[pl.ds(r0, _CHR)], pos_v)

        def batch_body(b, carry2):
            bb = b0 + b
            pltpu.sync_copy(in_hbm.at[bb, pl.ds(r0, _CHR)], buf_v)

            def row_body(r, carry3):
                for c in range(_NVC):
                    off = c * 16
                    buf_v[r, pl.ds(off, 16)] = (
                        buf_v[r, pl.ds(off, 16)] + pos_v[r, pl.ds(off, 16)]
                    )
                return carry3

            lax.fori_loop(0, _CHR, row_body, carry2)
            pltpu.sync_copy(buf_v, out_hbm.at[bb, pl.ds(r0, _CHR)])
            return carry2

        lax.fori_loop(0, _BPG, batch_body, carry)
        return carry

    lax.fori_loop(0, _NCH, chunk_body, 0)

    @pl.when(j == 0)
    def _():
        pltpu.sync_copy(cls_hbm.at[0], cls_v)
        pltpu.sync_copy(pos_hbm.at[pl.ds(_L, 1)], tmp_v)

        def cls_col(c, carry):
            off = c * 16
            cls_v[0, pl.ds(off, 16)] = (
                cls_v[0, pl.ds(off, 16)] + tmp_v[0, pl.ds(off, 16)]
            )
            return carry

        lax.fori_loop(0, _NVC, cls_col, 0)

        def cls_batch(b, carry):
            pltpu.sync_copy(cls_v, out_hbm.at[b0 + b, pl.ds(_L, 1)])
            return carry

        lax.fori_loop(0, _BPG, cls_batch, 0)


@functools.partial(
    pl.kernel,
    mesh=plsc.VectorSubcoreMesh(core_axis_name="c", subcore_axis_name="s"),
    out_type=jax.ShapeDtypeStruct((_B, _L + 1, _D), jnp.float32),
    scratch_types=[
        pltpu.VMEM((_CHR, _D), jnp.float32),
        pltpu.VMEM((_CHR, _D), jnp.float32),
        pltpu.VMEM((1, _D), jnp.float32),
        pltpu.VMEM((1, _D), jnp.float32),
    ],
)
def _sc_kernel(in_hbm, pos_hbm, cls_hbm, out_hbm, pos_v, buf_v, cls_v, tmp_v):
    _sc_body(in_hbm, pos_hbm, cls_hbm, out_hbm, pos_v, buf_v, cls_v, tmp_v)


def kernel(inputs, pos_table, class_token):
    return _sc_kernel(inputs, pos_table, class_token)
